# baseline (speedup 1.0000x reference)
import functools

import jax
import jax.numpy as jnp
from jax.experimental import pallas as pl
from jax.experimental.pallas import tpu as pltpu

_LANES = 128
_NEG = -3.0


def _knn_body(q_ref, s_ref, out_ref, sn_ref, *, n_way, m_real):
    m_pad = s_ref.shape[-1]

    @pl.when(pl.program_id(0) == 0)
    def _():
        s = s_ref[...]
        rs = 1.0 / (jnp.sqrt(jnp.sum(s * s, axis=1, keepdims=True)) + 1e-8)
        sn_ref[...] = s * rs

    g = q_ref.shape[0]
    hw = q_ref.shape[2]
    n_chunks = m_pad // _LANES
    n_real_last = m_real - _LANES * (n_chunks - 1)
    lane = jax.lax.broadcasted_iota(jnp.int32, (hw, _LANES), 1)
    per_query = []
    for i in range(g):
        qb = q_ref[i]
        rq = 1.0 / (jnp.sqrt(jnp.sum(qb * qb, axis=0, keepdims=True)) + 1e-8)
        qn = qb * rq
        per_class = []
        for j in range(n_way):
            inner = jax.lax.dot_general(
                qn, sn_ref[j],
                dimension_numbers=(((0,), (0,)), ((), ())),
                preferred_element_type=jnp.float32,
            )
            def chunk(c):
                v = inner[:, c * _LANES:(c + 1) * _LANES]
                if c == n_chunks - 1 and n_real_last < _LANES:
                    v = jnp.where(lane < n_real_last, v, _NEG)
                return v
            a = chunk(0)
            b = jnp.minimum(a, chunk(1))
            a = jnp.maximum(a, chunk(1))
            for c in range(2, n_chunks):
                v = chunk(c)
                a2 = jnp.maximum(a, v)
                t = jnp.minimum(a, v)
                b2 = jnp.maximum(b, t)
                u = jnp.minimum(b, t)
                if c == 2:
                    cc = u
                else:
                    cc = jnp.maximum(cc, u)
                a, b = a2, b2
            cand = jnp.concatenate([a, b, cc], axis=1)
            m1 = jnp.max(cand, axis=1, keepdims=True)
            eq1 = cand == m1
            n1 = jnp.sum(eq1.astype(jnp.float32), axis=1, keepdims=True)
            s2 = jnp.where(eq1, _NEG, cand)
            m2 = jnp.max(s2, axis=1, keepdims=True)
            eq2 = s2 == m2
            n2 = jnp.sum(eq2.astype(jnp.float32), axis=1, keepdims=True)
            s3 = jnp.where(eq2, _NEG, s2)
            m3 = jnp.max(s3, axis=1, keepdims=True)
            t1 = jnp.minimum(n1, 3.0)
            t2 = jnp.clip(3.0 - n1, 0.0, n2)
            t3 = jnp.maximum(3.0 - n1 - n2, 0.0)
            per_class.append(m1 * t1 + m2 * t2 + m3 * t3)
        cat = jnp.concatenate(per_class, axis=1)
        per_query.append(jnp.sum(cat, axis=0, keepdims=True))
    out_ref[...] = jnp.concatenate(per_query, axis=0)[None]


def kernel(q, S, av_num):
    B, C, h, w = q.shape
    n_way, _, M = S.shape
    hw = h * w
    m_pad = ((M + _LANES - 1) // _LANES) * _LANES
    qf = q.reshape(B, C, hw)
    Sp = jnp.pad(S, ((0, 0), (0, 0), (0, m_pad - M)))

    G = 15
    out = pl.pallas_call(
        functools.partial(_knn_body, n_way=n_way, m_real=M),
        grid=(B // G,),
        in_specs=[
            pl.BlockSpec((G, C, hw), lambda b: (b, 0, 0)),
            pl.BlockSpec((n_way, C, m_pad), lambda b: (0, 0, 0)),
        ],
        out_specs=pl.BlockSpec((1, G, n_way), lambda b: (b, 0, 0)),
        out_shape=jax.ShapeDtypeStruct((B // G, G, n_way), jnp.float32),
        scratch_shapes=[pltpu.VMEM((n_way, C, m_pad), jnp.float32)],
        compiler_params=pltpu.CompilerParams(
            dimension_semantics=("arbitrary",),
        ),
    )(qf, Sp)
    out = out.reshape(B, n_way)
    return (out, out)


# speedup vs baseline: 1.1361x; 1.1361x over previous
import functools

import jax
import jax.numpy as jnp
from jax.experimental import pallas as pl
from jax.experimental.pallas import tpu as pltpu

_LANES = 128
_NEG = -3.0


def _knn_body(q_ref, s_ref, out_ref, sn_ref, *, n_way, m_real):
    m_pad = s_ref.shape[-1]

    @pl.when(pl.program_id(0) == 0)
    def _():
        s = s_ref[...]
        rs = 1.0 / (jnp.sqrt(jnp.sum(s * s, axis=1, keepdims=True)) + 1e-8)
        sn_ref[...] = s * rs

    g = q_ref.shape[0]
    hw = q_ref.shape[2]
    n_chunks = m_pad // _LANES
    n_real_last = m_real - _LANES * (n_chunks - 1)
    lane = jax.lax.broadcasted_iota(jnp.int32, (hw, _LANES), 1)
    per_query = []
    for i in range(g):
        qb = q_ref[i]
        rq = 1.0 / (jnp.sqrt(jnp.sum(qb * qb, axis=0, keepdims=True)) + 1e-8)
        qn = qb * rq
        per_class = []
        for j in range(n_way):
            inner = jax.lax.dot_general(
                qn, sn_ref[j],
                dimension_numbers=(((0,), (0,)), ((), ())),
                preferred_element_type=jnp.float32,
            )
            def chunk(c):
                v = inner[:, c * _LANES:(c + 1) * _LANES]
                if c == n_chunks - 1 and n_real_last < _LANES:
                    v = jnp.where(lane < n_real_last, v, _NEG)
                return v
            a = chunk(0)
            b = jnp.minimum(a, chunk(1))
            a = jnp.maximum(a, chunk(1))
            for c in range(2, n_chunks):
                v = chunk(c)
                a2 = jnp.maximum(a, v)
                t = jnp.minimum(a, v)
                b2 = jnp.maximum(b, t)
                u = jnp.minimum(b, t)
                if c == 2:
                    cc = u
                else:
                    cc = jnp.maximum(cc, u)
                a, b = a2, b2
            cand = jnp.concatenate([a, b], axis=1)
            m1 = jnp.max(cand, axis=1, keepdims=True)
            tri = jnp.max(jnp.where(a == m1, a + b + cc, _NEG),
                          axis=1, keepdims=True)
            eq1 = cand == m1
            n1 = jnp.sum(eq1.astype(jnp.float32), axis=1, keepdims=True)
            s2 = jnp.where(eq1, _NEG, cand)
            m2 = jnp.max(s2, axis=1, keepdims=True)
            eq2 = s2 == m2
            n2 = jnp.sum(eq2.astype(jnp.float32), axis=1, keepdims=True)
            s3 = jnp.where(eq2, _NEG, s2)
            m3 = jnp.max(s3, axis=1, keepdims=True)
            t1 = jnp.minimum(n1, 3.0)
            t2 = jnp.clip(3.0 - n1, 0.0, n2)
            t3 = jnp.maximum(3.0 - n1 - n2, 0.0)
            top3 = jnp.maximum(m1 * t1 + m2 * t2 + m3 * t3, tri)
            per_class.append(top3)
        cat = jnp.concatenate(per_class, axis=1)
        per_query.append(jnp.sum(cat, axis=0, keepdims=True))
    out_ref[...] = jnp.concatenate(per_query, axis=0)[None]


def kernel(q, S, av_num):
    B, C, h, w = q.shape
    n_way, _, M = S.shape
    hw = h * w
    m_pad = ((M + _LANES - 1) // _LANES) * _LANES
    qf = q.reshape(B, C, hw)
    Sp = jnp.pad(S, ((0, 0), (0, 0), (0, m_pad - M)))

    G = 5
    out = pl.pallas_call(
        functools.partial(_knn_body, n_way=n_way, m_real=M),
        grid=(B // G,),
        in_specs=[
            pl.BlockSpec((G, C, hw), lambda b: (b, 0, 0)),
            pl.BlockSpec((n_way, C, m_pad), lambda b: (0, 0, 0)),
        ],
        out_specs=pl.BlockSpec((1, G, n_way), lambda b: (b, 0, 0)),
        out_shape=jax.ShapeDtypeStruct((B // G, G, n_way), jnp.float32),
        scratch_shapes=[pltpu.VMEM((n_way, C, m_pad), jnp.float32)],
        compiler_params=pltpu.CompilerParams(
            dimension_semantics=("arbitrary",),
        ),
    )(qf, Sp)
    out = out.reshape(B, n_way)
    return (out, out)


# speedup vs baseline: 1.1682x; 1.0282x over previous
import functools

import jax
import jax.numpy as jnp
from jax.experimental import pallas as pl
from jax.experimental.pallas import tpu as pltpu

_LANES = 128
_NEG = -3.0


def _knn_body(q_ref, s_ref, out_ref, sn_ref, *, n_way, m_real):
    m_pad = s_ref.shape[-1]

    @pl.when(pl.program_id(0) == 0)
    def _():
        s = s_ref[...]
        rs = 1.0 / (jnp.sqrt(jnp.sum(s * s, axis=1, keepdims=True)) + 1e-8)
        sn_ref[...] = s * rs

    g = q_ref.shape[0]
    hw = q_ref.shape[2]
    n_chunks = m_pad // _LANES
    n_real_last = m_real - _LANES * (n_chunks - 1)
    lane = jax.lax.broadcasted_iota(jnp.int32, (hw, _LANES), 1)
    per_query = []
    for i in range(g):
        qb = q_ref[i]
        rq = 1.0 / (jnp.sqrt(jnp.sum(qb * qb, axis=0, keepdims=True)) + 1e-8)
        qn = qb * rq
        per_class = []
        for j in range(n_way):
            inner = jax.lax.dot_general(
                qn, sn_ref[j],
                dimension_numbers=(((0,), (0,)), ((), ())),
                preferred_element_type=jnp.float32,
            )
            def chunk(c):
                v = inner[:, c * _LANES:(c + 1) * _LANES]
                if c == n_chunks - 1 and n_real_last < _LANES:
                    v = jnp.where(lane < n_real_last, v, _NEG)
                return v
            a = chunk(0)
            b = jnp.minimum(a, chunk(1))
            a = jnp.maximum(a, chunk(1))
            for c in range(2, n_chunks):
                v = chunk(c)
                a2 = jnp.maximum(a, v)
                t = jnp.minimum(a, v)
                b2 = jnp.maximum(b, t)
                u = jnp.minimum(b, t)
                if c == 2:
                    cc = u
                else:
                    cc = jnp.maximum(cc, u)
                a, b = a2, b2
            cand = jnp.concatenate([a, b, cc], axis=1)
            m1 = jnp.max(cand, axis=1, keepdims=True)
            eq1 = cand == m1
            n1 = jnp.sum(eq1.astype(jnp.float32), axis=1, keepdims=True)
            s2 = jnp.where(eq1, _NEG, cand)
            m2 = jnp.max(s2, axis=1, keepdims=True)
            eq2 = s2 == m2
            n2 = jnp.sum(eq2.astype(jnp.float32), axis=1, keepdims=True)
            s3 = jnp.where(eq2, _NEG, s2)
            m3 = jnp.max(s3, axis=1, keepdims=True)
            t1 = jnp.minimum(n1, 3.0)
            t2 = jnp.clip(3.0 - n1, 0.0, n2)
            t3 = jnp.maximum(3.0 - n1 - n2, 0.0)
            per_class.append(m1 * t1 + m2 * t2 + m3 * t3)
        cat = jnp.concatenate(per_class, axis=1)
        per_query.append(jnp.sum(cat, axis=0, keepdims=True))
    out_ref[...] = jnp.concatenate(per_query, axis=0)[None]


def kernel(q, S, av_num):
    B, C, h, w = q.shape
    n_way, _, M = S.shape
    hw = h * w
    m_pad = ((M + _LANES - 1) // _LANES) * _LANES
    qf = q.reshape(B, C, hw)
    Sp = jnp.pad(S, ((0, 0), (0, 0), (0, m_pad - M)))

    G = 5
    out = pl.pallas_call(
        functools.partial(_knn_body, n_way=n_way, m_real=M),
        grid=(B // G,),
        in_specs=[
            pl.BlockSpec((G, C, hw), lambda b: (b, 0, 0)),
            pl.BlockSpec((n_way, C, m_pad), lambda b: (0, 0, 0)),
        ],
        out_specs=pl.BlockSpec((1, G, n_way), lambda b: (b, 0, 0)),
        out_shape=jax.ShapeDtypeStruct((B // G, G, n_way), jnp.float32),
        scratch_shapes=[pltpu.VMEM((n_way, C, m_pad), jnp.float32)],
        compiler_params=pltpu.CompilerParams(
            dimension_semantics=("arbitrary",),
        ),
    )(qf, Sp)
    out = out.reshape(B, n_way)
    return (out, out)


# speedup vs baseline: 1.1684x; 1.0001x over previous
import functools

import jax
import jax.numpy as jnp
from jax.experimental import pallas as pl
from jax.experimental.pallas import tpu as pltpu

_LANES = 128
_NEG = -3.0


def _knn_body(q_ref, s_ref, out_ref, sn_ref, *, n_way, m_real):
    m_pad = s_ref.shape[-1]

    @pl.when(pl.program_id(0) == 0)
    def _():
        s = s_ref[...]
        rs = 1.0 / (jnp.sqrt(jnp.sum(s * s, axis=1, keepdims=True)) + 1e-8)
        sn_ref[...] = s * rs

    g = q_ref.shape[0]
    hw = q_ref.shape[2]
    n_chunks = m_pad // _LANES
    n_real_last = m_real - _LANES * (n_chunks - 1)
    lane = jax.lax.broadcasted_iota(jnp.int32, (hw, _LANES), 1)
    per_query = []
    for i in range(g):
        qb = q_ref[i]
        rq = 1.0 / (jnp.sqrt(jnp.sum(qb * qb, axis=0, keepdims=True)) + 1e-8)
        qn = qb * rq
        per_class = []
        for j in range(n_way):
            inner = jax.lax.dot_general(
                qn, sn_ref[j],
                dimension_numbers=(((0,), (0,)), ((), ())),
                preferred_element_type=jnp.float32,
            )
            def chunk(c):
                v = inner[:, c * _LANES:(c + 1) * _LANES]
                if c == n_chunks - 1 and n_real_last < _LANES:
                    v = jnp.where(lane < n_real_last, v, _NEG)
                return v
            a = chunk(0)
            b = jnp.minimum(a, chunk(1))
            a = jnp.maximum(a, chunk(1))
            for c in range(2, n_chunks):
                v = chunk(c)
                a2 = jnp.maximum(a, v)
                t = jnp.minimum(a, v)
                b2 = jnp.maximum(b, t)
                u = jnp.minimum(b, t)
                if c == 2:
                    cc = u
                else:
                    cc = jnp.maximum(cc, u)
                a, b = a2, b2
            cand = jnp.concatenate([a, b, cc], axis=1)
            m1 = jnp.max(cand, axis=1, keepdims=True)
            eq1 = cand == m1
            n1 = jnp.sum(eq1.astype(jnp.float32), axis=1, keepdims=True)
            s2 = jnp.where(eq1, _NEG, cand)
            m2 = jnp.max(s2, axis=1, keepdims=True)
            eq2 = s2 == m2
            n2 = jnp.sum(eq2.astype(jnp.float32), axis=1, keepdims=True)
            s3 = jnp.where(eq2, _NEG, s2)
            m3 = jnp.max(s3, axis=1, keepdims=True)
            t1 = jnp.minimum(n1, 3.0)
            t2 = jnp.clip(3.0 - n1, 0.0, n2)
            t3 = jnp.maximum(3.0 - n1 - n2, 0.0)
            per_class.append(m1 * t1 + m2 * t2 + m3 * t3)
        cat = jnp.concatenate(per_class, axis=1)
        per_query.append(jnp.sum(cat, axis=0, keepdims=True))
    out_ref[...] = jnp.concatenate(per_query, axis=0)[None]


def kernel(q, S, av_num):
    B, C, h, w = q.shape
    n_way, _, M = S.shape
    hw = h * w
    m_pad = ((M + _LANES - 1) // _LANES) * _LANES
    qf = q.reshape(B, C, hw)
    Sp = jnp.pad(S, ((0, 0), (0, 0), (0, m_pad - M)))

    G = next(g for g in (5, 3, 1) if B % g == 0)
    out = pl.pallas_call(
        functools.partial(_knn_body, n_way=n_way, m_real=M),
        grid=(B // G,),
        in_specs=[
            pl.BlockSpec((G, C, hw), lambda b: (b, 0, 0)),
            pl.BlockSpec((n_way, C, m_pad), lambda b: (0, 0, 0)),
        ],
        out_specs=pl.BlockSpec((1, G, n_way), lambda b: (b, 0, 0)),
        out_shape=jax.ShapeDtypeStruct((B // G, G, n_way), jnp.float32),
        scratch_shapes=[pltpu.VMEM((n_way, C, m_pad), jnp.float32)],
        compiler_params=pltpu.CompilerParams(
            dimension_semantics=("arbitrary",),
        ),
    )(qf, Sp)
    out = out.reshape(B, n_way)
    return (out, out)
